# Initial kernel scaffold; baseline (speedup 1.0000x reference)
import jax
import jax.numpy as jnp
from jax.experimental import pallas as pl


def kernel(x, weight):
    raise NotImplementedError("write your pallas kernel here")


# speedup vs baseline: 6.3421x; 6.3421x over previous
import functools

import jax
import jax.numpy as jnp
from jax import lax
from jax.experimental import pallas as pl
from jax.experimental.pallas import tpu as pltpu
from jax.experimental.pallas import tpu_sc as plsc

_EMB_D = 128
_CHUNK = 128


def _sc_gather(weight, idx2d):
    n_rows, _ = idx2d.shape
    info = plsc.get_sparse_core_info()
    nw = info.num_cores * info.num_subcores
    nc = n_rows // nw
    mesh = plsc.VectorSubcoreMesh(core_axis_name="c", subcore_axis_name="s")

    @functools.partial(
        pl.kernel,
        mesh=mesh,
        out_type=jax.ShapeDtypeStruct((n_rows * _CHUNK, _EMB_D), jnp.float32),
        scratch_types=[
            pltpu.VMEM((nc, _CHUNK), jnp.int32),
            pltpu.VMEM((_CHUNK, _EMB_D), jnp.float32),
            pltpu.SemaphoreType.DMA,
        ],
    )
    def k(table_hbm, idx_hbm, out_hbm, idx_v, rows_v, sem):
        wid = lax.axis_index("s") * info.num_cores + lax.axis_index("c")
        pltpu.sync_copy(idx_hbm.at[pl.ds(wid * nc, nc)], idx_v)

        def body(g, carry):
            pltpu.async_copy(table_hbm.at[idx_v.at[g]], rows_v, sem).wait()
            base = pl.multiple_of((wid * nc + g) * _CHUNK, _CHUNK)
            pltpu.sync_copy(rows_v, out_hbm.at[pl.ds(base, _CHUNK)])
            return carry

        lax.fori_loop(0, nc, body, 0)

    return k(weight, idx2d)


def kernel(x, weight):
    flat = x.reshape(-1).astype(jnp.int32)
    idx2d = flat.reshape(-1, _CHUNK)
    out = _sc_gather(weight, idx2d)
    return out.reshape(x.shape + (_EMB_D,))


# speedup vs baseline: 9.1793x; 1.4474x over previous
import functools

import jax
import jax.numpy as jnp
from jax import lax
from jax.experimental import pallas as pl
from jax.experimental.pallas import tpu as pltpu
from jax.experimental.pallas import tpu_sc as plsc

_EMB_D = 128
_CHUNK = 128
_NBUF = 4


def _sc_gather(weight, idx2d):
    n_rows, _ = idx2d.shape
    info = plsc.get_sparse_core_info()
    nw = info.num_cores * info.num_subcores
    nc = n_rows // nw
    mesh = plsc.VectorSubcoreMesh(core_axis_name="c", subcore_axis_name="s")

    @functools.partial(
        pl.kernel,
        mesh=mesh,
        out_type=jax.ShapeDtypeStruct((n_rows * _CHUNK, _EMB_D), jnp.float32),
        scratch_types=(
            [pltpu.VMEM((nc, _CHUNK), jnp.int32)]
            + [pltpu.VMEM((_CHUNK, _EMB_D), jnp.float32)] * _NBUF
            + [pltpu.SemaphoreType.DMA] * (2 * _NBUF)
        ),
    )
    def k(table_hbm, idx_hbm, out_hbm, idx_v, *bufs_and_sems):
        rows = bufs_and_sems[:_NBUF]
        gsem = bufs_and_sems[_NBUF:2 * _NBUF]
        wsem = bufs_and_sems[2 * _NBUF:]
        wid = lax.axis_index("s") * info.num_cores + lax.axis_index("c")
        pltpu.sync_copy(idx_hbm.at[pl.ds(wid * nc, nc)], idx_v)

        def gather(b, g):
            pltpu.make_async_copy(
                table_hbm.at[idx_v.at[g]], rows[b], gsem[b]).start()

        for b in range(_NBUF):
            gather(b, b)

        def outer(k_, carry):
            for b in range(_NBUF):
                g = k_ * _NBUF + b
                pltpu.make_async_copy(
                    table_hbm.at[idx_v.at[g]], rows[b], gsem[b]).wait()
                base = pl.multiple_of((wid * nc + g) * _CHUNK, _CHUNK)
                out_slice = out_hbm.at[pl.ds(base, _CHUNK)]
                cp = pltpu.make_async_copy(rows[b], out_slice, wsem[b])
                cp.start()
                cp.wait()

                @pl.when(g + _NBUF < nc)
                def _():
                    gather(b, g + _NBUF)
            return carry

        lax.fori_loop(0, nc // _NBUF, outer, 0)

    return k(weight, idx2d)


def kernel(x, weight):
    flat = x.reshape(-1).astype(jnp.int32)
    idx2d = flat.reshape(-1, _CHUNK)
    out = _sc_gather(weight, idx2d)
    return out.reshape(x.shape + (_EMB_D,))


# speedup vs baseline: 9.1808x; 1.0002x over previous
import functools

import jax
import jax.numpy as jnp
from jax import lax
from jax.experimental import pallas as pl
from jax.experimental.pallas import tpu as pltpu
from jax.experimental.pallas import tpu_sc as plsc

_EMB_D = 128
_CHUNK = 128
_NBUF = 5


def _sc_gather(weight, idx2d):
    n_rows, _ = idx2d.shape
    info = plsc.get_sparse_core_info()
    nw = info.num_cores * info.num_subcores
    nc = n_rows // nw
    mesh = plsc.VectorSubcoreMesh(core_axis_name="c", subcore_axis_name="s")

    @functools.partial(
        pl.kernel,
        mesh=mesh,
        out_type=jax.ShapeDtypeStruct((n_rows * _CHUNK, _EMB_D), jnp.float32),
        scratch_types=(
            [pltpu.VMEM((nc, _CHUNK), jnp.int32)]
            + [pltpu.VMEM((_CHUNK, _EMB_D), jnp.float32)] * _NBUF
            + [pltpu.SemaphoreType.DMA] * (2 * _NBUF)
        ),
    )
    def k(table_hbm, idx_hbm, out_hbm, idx_v, *bufs_and_sems):
        rows = bufs_and_sems[:_NBUF]
        gsem = bufs_and_sems[_NBUF:2 * _NBUF]
        wsem = bufs_and_sems[2 * _NBUF:]
        wid = lax.axis_index("s") * info.num_cores + lax.axis_index("c")
        pltpu.sync_copy(idx_hbm.at[pl.ds(wid * nc, nc)], idx_v)

        def gather(b, g):
            pltpu.make_async_copy(
                table_hbm.at[idx_v.at[g]], rows[b], gsem[b]).start()

        for b in range(_NBUF):
            gather(b, b)

        def outer(k_, carry):
            for b in range(_NBUF):
                g = k_ * _NBUF + b
                pltpu.make_async_copy(
                    table_hbm.at[idx_v.at[g]], rows[b], gsem[b]).wait()
                base = pl.multiple_of((wid * nc + g) * _CHUNK, _CHUNK)
                out_slice = out_hbm.at[pl.ds(base, _CHUNK)]
                cp = pltpu.make_async_copy(rows[b], out_slice, wsem[b])
                cp.start()
                cp.wait()

                @pl.when(g + _NBUF < nc)
                def _():
                    gather(b, g + _NBUF)
            return carry

        lax.fori_loop(0, nc // _NBUF, outer, 0)

    return k(weight, idx2d)


def kernel(x, weight):
    flat = x.reshape(-1).astype(jnp.int32)
    idx2d = flat.reshape(-1, _CHUNK)
    out = _sc_gather(weight, idx2d)
    return out.reshape(x.shape + (_EMB_D,))


# speedup vs baseline: 9.1827x; 1.0002x over previous
import functools

import jax
import jax.numpy as jnp
from jax import lax
from jax.experimental import pallas as pl
from jax.experimental.pallas import tpu as pltpu
from jax.experimental.pallas import tpu_sc as plsc

_EMB_D = 128
_CHUNK = 128
_NBUF = 4


def _sc_gather(weight, idx2d):
    n_rows, _ = idx2d.shape
    info = plsc.get_sparse_core_info()
    nw = info.num_cores * info.num_subcores
    nc = n_rows // nw
    mesh = plsc.VectorSubcoreMesh(core_axis_name="c", subcore_axis_name="s")

    @functools.partial(
        pl.kernel,
        mesh=mesh,
        out_type=jax.ShapeDtypeStruct((n_rows * _CHUNK, _EMB_D), jnp.float32),
        scratch_types=(
            [pltpu.VMEM((nc, _CHUNK), jnp.int32)]
            + [pltpu.VMEM((_CHUNK, _EMB_D), jnp.float32)] * _NBUF
            + [pltpu.SemaphoreType.DMA] * (2 * _NBUF)
        ),
    )
    def k(table_hbm, idx_hbm, out_hbm, idx_v, *bufs_and_sems):
        rows = bufs_and_sems[:_NBUF]
        gsem = bufs_and_sems[_NBUF:2 * _NBUF]
        wsem = bufs_and_sems[2 * _NBUF:]
        wid = lax.axis_index("s") * info.num_cores + lax.axis_index("c")
        pltpu.sync_copy(idx_hbm.at[pl.ds(wid * nc, nc)], idx_v)

        def gather(b, g):
            pltpu.make_async_copy(
                table_hbm.at[idx_v.at[g]], rows[b], gsem[b]).start()

        for b in range(_NBUF):
            gather(b, b)

        def outer(k_, carry):
            for b in range(_NBUF):
                g = k_ * _NBUF + b
                pltpu.make_async_copy(
                    table_hbm.at[idx_v.at[g]], rows[b], gsem[b]).wait()
                base = pl.multiple_of((wid * nc + g) * _CHUNK, _CHUNK)
                out_slice = out_hbm.at[pl.ds(base, _CHUNK)]
                cp = pltpu.make_async_copy(rows[b], out_slice, wsem[b])
                cp.start()
                cp.wait()

                @pl.when(g + _NBUF < nc)
                def _():
                    gather(b, g + _NBUF)
            return carry

        lax.fori_loop(0, nc // _NBUF, outer, 0)

    return k(weight, idx2d)


def kernel(x, weight):
    flat = x.reshape(-1).astype(jnp.int32)
    idx2d = flat.reshape(-1, _CHUNK)
    out = _sc_gather(weight, idx2d)
    return out.reshape(x.shape + (_EMB_D,))
